# baseline (speedup 1.0000x reference)
import functools

import jax
import jax.numpy as jnp
from jax import lax
from jax.experimental import pallas as pl
from jax.experimental.pallas import tpu as pltpu
from jax.experimental.pallas import tpu_sc as plsc

N_NODES = 10000
N_PAD = 10240
N_EDGES = 320000
D = 128
DE = 16
G = 256

NC, NS = 2, 16
NW = NC * NS
CH = 80
EPW = N_EDGES // NW
NCH = EPW // CH
RPT = N_PAD // NS

BN = 2000
NB = N_NODES // BN

_MESH = plsc.VectorSubcoreMesh(core_axis_name="core", subcore_axis_name="subcore")
_SC_PARAMS = pltpu.CompilerParams(use_tc_tiling_on_sc=False)
_SC_PARAMS_NL = pltpu.CompilerParams(use_tc_tiling_on_sc=False,
                                     needs_layout_passes=False)


def _zero_rows(buf, n, w):
    @pl.loop(0, n)
    def _(i):
        @pl.loop(0, w, step=16)
        def _(j):
            buf[i, pl.ds(j, 16)] = jnp.zeros((16,), jnp.float32)


def _scatter_phase(dummy_hbm, idx_v, acc_s, b0, b1, sem0, sem1, gather):
    def wait(buf, sem):
        pltpu.make_async_copy(dummy_hbm, buf, sem).wait()

    gather(0, b0, sem0)

    @pl.loop(0, (NCH - 1) // 2)
    def _(t):
        j = 2 * t
        wait(b0, sem0)
        gather(j + 1, b1, sem1)
        pltpu.sync_copy(b0, acc_s.at[idx_v.at[j]], add=True)
        wait(b1, sem1)
        gather(j + 2, b0, sem0)
        pltpu.sync_copy(b1, acc_s.at[idx_v.at[j + 1]], add=True)

    wait(b0, sem0)
    pltpu.sync_copy(b0, acc_s.at[idx_v.at[NCH - 1]], add=True)


def _sc_segment_sum_x(x, src3, dst3):
    @functools.partial(
        pl.kernel,
        out_type=jax.ShapeDtypeStruct((NC, N_PAD, D), jnp.float32),
        mesh=_MESH,
        compiler_params=_SC_PARAMS,
        scratch_types=[
            pltpu.VMEM((NCH, CH), jnp.int32),
            pltpu.VMEM((NCH, CH), jnp.int32),
            pltpu.VMEM((CH, D), jnp.float32),
            pltpu.VMEM((CH, D), jnp.float32),
            pltpu.VMEM_SHARED((N_PAD, D), jnp.float32),
            pltpu.SemaphoreType.DMA,
            pltpu.SemaphoreType.DMA,
        ],
    )
    def k(x_hbm, src_hbm, dst_hbm, out_hbm,
          src_v, dst_v, b0, b1, acc_s, sem0, sem1):
        cid = lax.axis_index("core")
        sid = lax.axis_index("subcore")
        wid = cid * NS + sid

        _zero_rows(b0, CH, D)
        row0 = sid * RPT
        for kk in range(RPT // CH):
            pltpu.sync_copy(b0, acc_s.at[pl.ds(row0 + kk * CH, CH)])
        plsc.subcore_barrier()

        pltpu.sync_copy(src_hbm.at[wid], src_v)
        pltpu.sync_copy(dst_hbm.at[wid], dst_v)

        def gather(j, buf, sem):
            pltpu.async_copy(x_hbm.at[src_v.at[j]], buf, sem)

        _scatter_phase(x_hbm.at[pl.ds(0, CH)], dst_v, acc_s,
                       b0, b1, sem0, sem1, gather)

        plsc.subcore_barrier()
        pltpu.sync_copy(acc_s.at[pl.ds(row0, RPT)],
                        out_hbm.at[cid, pl.ds(row0, RPT)])

    return k(x, src3, dst3)


NEH = 4
NWH = NS // 2


def _sc_segment_sum_ea(eaT, dst3):
    @functools.partial(
        pl.kernel,
        out_type=jax.ShapeDtypeStruct((NEH, DE, N_PAD), jnp.float32),
        mesh=_MESH,
        compiler_params=_SC_PARAMS_NL,
        scratch_types=[
            pltpu.VMEM((NCH, CH), jnp.int32),
            pltpu.VMEM((NCH, CH), jnp.int32),
            pltpu.VMEM((EPW,), jnp.float32),
            pltpu.VMEM((EPW,), jnp.float32),
            pltpu.VMEM((EPW,), jnp.float32),
            pltpu.VMEM((EPW,), jnp.float32),
            pltpu.VMEM((N_PAD,), jnp.float32),
            pltpu.VMEM((N_PAD,), jnp.float32),
            pltpu.SemaphoreType.DMA,
            pltpu.SemaphoreType.DMA,
            pltpu.SemaphoreType.DMA,
            pltpu.SemaphoreType.DMA,
        ],
    )
    def k(eaT_hbm, dst_hbm, out_hbm, d0, d1, va0, va1, vb0, vb1,
          acca, accb, sd0, sd1, sv0, sv1):
        cid = lax.axis_index("core")
        sid = lax.axis_index("subcore")
        q = cid * 2 + sid // 8
        fa = sid % 8
        w0 = q * NWH

        @pl.loop(0, N_PAD, step=16)
        def _(i):
            acca[pl.ds(i, 16)] = jnp.zeros((16,), jnp.float32)
            accb[pl.ds(i, 16)] = jnp.zeros((16,), jnp.float32)

        def load_w(w, db, va, vb, sd, sv):
            pltpu.async_copy(dst_hbm.at[w0 + w], db, sd)
            ecol = (w0 + w) * EPW
            pltpu.async_copy(eaT_hbm.at[fa, pl.ds(ecol, EPW)], va, sv)
            pltpu.async_copy(eaT_hbm.at[fa + 8, pl.ds(ecol, EPW)], vb, sv)

        def wait_w(db, va, vb, sd, sv):
            pltpu.make_async_copy(dst_hbm.at[0], db, sd).wait()
            pltpu.make_async_copy(eaT_hbm.at[0, pl.ds(0, EPW)], va, sv).wait()
            pltpu.make_async_copy(eaT_hbm.at[0, pl.ds(0, EPW)], vb, sv).wait()

        def accumulate(db, va, vb):
            @pl.loop(0, NCH, step=5)
            def _(r0):
                for dr in range(5):
                    for c5 in range(CH // 16):
                        idx = db[r0 + dr, pl.ds(c5 * 16, 16)]
                        off = (r0 + dr) * CH + c5 * 16
                        plsc.addupdate_scatter(acca, [idx],
                                               va[pl.ds(off, 16)])
                        plsc.addupdate_scatter(accb, [idx],
                                               vb[pl.ds(off, 16)])

        load_w(0, d0, va0, vb0, sd0, sv0)

        @pl.loop(0, NWH // 2)
        def _(t):
            w = 2 * t
            wait_w(d0, va0, vb0, sd0, sv0)
            load_w(w + 1, d1, va1, vb1, sd1, sv1)
            accumulate(d0, va0, vb0)
            wait_w(d1, va1, vb1, sd1, sv1)

            @pl.when(w + 2 < NWH)
            def _():
                load_w(w + 2, d0, va0, vb0, sd0, sv0)

            accumulate(d1, va1, vb1)

        pltpu.sync_copy(acca, out_hbm.at[q, fa])
        pltpu.sync_copy(accb, out_hbm.at[q, fa + 8])

    return k(eaT, dst3)


def _tc_mlp_pool(x, aggx, agge, batch3, W_edge, W1, b1r, W2, b2r):
    hi = lax.Precision.HIGHEST

    def mm(a, b):
        return lax.dot_general(a, b, (((1,), (0,)), ((), ())),
                               precision=hi,
                               preferred_element_type=jnp.float32)

    def body(x_r, a0_r, a1_r, e0_r, e1_r, e2_r, e3_r, we_r, w1_r, b1_r,
             w2_r, b2_r, bat_r, nr_r, gr_r, sums, counts):
        i = pl.program_id(0)

        @pl.when(i == 0)
        def _():
            sums[...] = jnp.zeros_like(sums)
            counts[...] = jnp.zeros_like(counts)

        a16 = (e0_r[0] + e1_r[0]) + (e2_r[0] + e3_r[0])
        h0 = x_r[...] + a0_r[0] + a1_r[0] + mm(a16, we_r[...])
        h = jnp.maximum(mm(h0, w1_r[...]) + b1_r[...], 0.0)
        nr = mm(h, w2_r[...]) + b2_r[...]
        nr_r[...] = nr

        seg = bat_r[0]
        P = (lax.broadcasted_iota(jnp.int32, (G, BN), 0) == seg)
        P = P.astype(jnp.float32)
        sums[...] += mm(P, nr)
        counts[...] += jnp.sum(P, axis=1, keepdims=True)

        @pl.when(i == NB - 1)
        def _():
            gr_r[...] = sums[...] / jnp.maximum(counts[...], 1.0)

    nr, gr = pl.pallas_call(
        body,
        grid=(NB,),
        in_specs=[
            pl.BlockSpec((BN, D), lambda i: (i, 0)),
            pl.BlockSpec((1, BN, D), lambda i: (0, i, 0)),
            pl.BlockSpec((1, BN, D), lambda i: (1, i, 0)),
            pl.BlockSpec((1, BN, DE), lambda i: (0, i, 0)),
            pl.BlockSpec((1, BN, DE), lambda i: (1, i, 0)),
            pl.BlockSpec((1, BN, DE), lambda i: (2, i, 0)),
            pl.BlockSpec((1, BN, DE), lambda i: (3, i, 0)),
            pl.BlockSpec((DE, D), lambda i: (0, 0)),
            pl.BlockSpec((D, 2 * D), lambda i: (0, 0)),
            pl.BlockSpec((1, 2 * D), lambda i: (0, 0)),
            pl.BlockSpec((2 * D, D), lambda i: (0, 0)),
            pl.BlockSpec((1, D), lambda i: (0, 0)),
            pl.BlockSpec((1, 1, BN), lambda i: (i, 0, 0)),
        ],
        out_specs=[
            pl.BlockSpec((BN, D), lambda i: (i, 0)),
            pl.BlockSpec((G, D), lambda i: (0, 0)),
        ],
        out_shape=[
            jax.ShapeDtypeStruct((N_NODES, D), jnp.float32),
            jax.ShapeDtypeStruct((G, D), jnp.float32),
        ],
        scratch_shapes=[
            pltpu.VMEM((G, D), jnp.float32),
            pltpu.VMEM((G, 1), jnp.float32),
        ],
        compiler_params=pltpu.CompilerParams(
            dimension_semantics=("arbitrary",)),
    )(x, aggx, aggx, agge, agge, agge, agge, W_edge, W1, b1r, W2, b2r, batch3)
    return gr, nr


def kernel(x, edge_index, edge_attr, batch, W_edge, W1, b1, W2, b2):
    src3 = edge_index[0].reshape(NW, NCH, CH)
    dst3 = edge_index[1].reshape(NW, NCH, CH)
    aggx = _sc_segment_sum_x(x, src3, dst3)
    agge = _sc_segment_sum_ea(edge_attr.T, dst3).transpose(0, 2, 1)
    batch3 = batch.reshape(NB, 1, BN)
    gr, nr = _tc_mlp_pool(x, aggx, agge, batch3, W_edge, W1,
                          b1.reshape(1, 2 * D), W2, b2.reshape(1, D))
    return (gr, nr)


# speedup vs baseline: 1.0962x; 1.0962x over previous
import functools

import jax
import jax.numpy as jnp
from jax import lax
from jax.experimental import pallas as pl
from jax.experimental.pallas import tpu as pltpu
from jax.experimental.pallas import tpu_sc as plsc

N_NODES = 10000
N_PAD = 10240
N_EDGES = 320000
D = 128
DE = 16
G = 256

NC, NS = 2, 16
NW = NC * NS
CH = 80
EPW = N_EDGES // NW
NCH = EPW // CH
RPT = N_PAD // NS

BN = 2000
NB = N_NODES // BN

_MESH = plsc.VectorSubcoreMesh(core_axis_name="core", subcore_axis_name="subcore")
_SC_PARAMS = pltpu.CompilerParams(use_tc_tiling_on_sc=False)
_SC_PARAMS_NL = pltpu.CompilerParams(use_tc_tiling_on_sc=False,
                                     needs_layout_passes=False)


def _zero_rows(buf, n, w):
    @pl.loop(0, n)
    def _(i):
        @pl.loop(0, w, step=16)
        def _(j):
            buf[i, pl.ds(j, 16)] = jnp.zeros((16,), jnp.float32)


def _scatter_phase(dummy_hbm, idx_v, acc_s, b0, b1, sem0, sem1, gather):
    def wait(buf, sem):
        pltpu.make_async_copy(dummy_hbm, buf, sem).wait()

    gather(0, b0, sem0)

    @pl.loop(0, (NCH - 1) // 2)
    def _(t):
        j = 2 * t
        wait(b0, sem0)
        gather(j + 1, b1, sem1)
        pltpu.sync_copy(b0, acc_s.at[idx_v.at[j]], add=True)
        wait(b1, sem1)
        gather(j + 2, b0, sem0)
        pltpu.sync_copy(b1, acc_s.at[idx_v.at[j + 1]], add=True)

    wait(b0, sem0)
    pltpu.sync_copy(b0, acc_s.at[idx_v.at[NCH - 1]], add=True)


def _sc_segment_sum_x(x, src3, dst3):
    @functools.partial(
        pl.kernel,
        out_type=jax.ShapeDtypeStruct((NC, N_PAD, D), jnp.float32),
        mesh=_MESH,
        compiler_params=_SC_PARAMS,
        scratch_types=[
            pltpu.VMEM((NCH, CH), jnp.int32),
            pltpu.VMEM((NCH, CH), jnp.int32),
            pltpu.VMEM((CH, D), jnp.float32),
            pltpu.VMEM((CH, D), jnp.float32),
            pltpu.VMEM_SHARED((N_PAD, D), jnp.float32),
            pltpu.SemaphoreType.DMA,
            pltpu.SemaphoreType.DMA,
        ],
    )
    def k(x_hbm, src_hbm, dst_hbm, out_hbm,
          src_v, dst_v, b0, b1, acc_s, sem0, sem1):
        cid = lax.axis_index("core")
        sid = lax.axis_index("subcore")
        wid = cid * NS + sid

        _zero_rows(b0, CH, D)
        row0 = sid * RPT
        for kk in range(RPT // CH):
            pltpu.sync_copy(b0, acc_s.at[pl.ds(row0 + kk * CH, CH)])
        plsc.subcore_barrier()

        pltpu.sync_copy(src_hbm.at[wid], src_v)
        pltpu.sync_copy(dst_hbm.at[wid], dst_v)

        def gather(j, buf, sem):
            pltpu.async_copy(x_hbm.at[src_v.at[j]], buf, sem)

        _scatter_phase(x_hbm.at[pl.ds(0, CH)], dst_v, acc_s,
                       b0, b1, sem0, sem1, gather)

        plsc.subcore_barrier()
        pltpu.sync_copy(acc_s.at[pl.ds(row0, RPT)],
                        out_hbm.at[cid, pl.ds(row0, RPT)])

    return k(x, src3, dst3)


NEH = 4
NWH = NS // 2


def _sc_segment_sum_ea(eaT, dst3):
    @functools.partial(
        pl.kernel,
        out_type=jax.ShapeDtypeStruct((NEH, DE, N_PAD), jnp.float32),
        mesh=_MESH,
        compiler_params=_SC_PARAMS_NL,
        scratch_types=[
            pltpu.VMEM((NCH, CH), jnp.int32),
            pltpu.VMEM((NCH, CH), jnp.int32),
            pltpu.VMEM((EPW,), jnp.float32),
            pltpu.VMEM((EPW,), jnp.float32),
            pltpu.VMEM((EPW,), jnp.float32),
            pltpu.VMEM((EPW,), jnp.float32),
            pltpu.VMEM((N_PAD,), jnp.float32),
            pltpu.VMEM((N_PAD,), jnp.float32),
            pltpu.SemaphoreType.DMA,
            pltpu.SemaphoreType.DMA,
            pltpu.SemaphoreType.DMA,
            pltpu.SemaphoreType.DMA,
        ],
    )
    def k(eaT_hbm, dst_hbm, out_hbm, d0, d1, va0, va1, vb0, vb1,
          acca, accb, sd0, sd1, sv0, sv1):
        cid = lax.axis_index("core")
        sid = lax.axis_index("subcore")
        q = cid * 2 + sid // 8
        fa = sid % 8
        w0 = q * NWH

        @pl.loop(0, N_PAD, step=16)
        def _(i):
            acca[pl.ds(i, 16)] = jnp.zeros((16,), jnp.float32)
            accb[pl.ds(i, 16)] = jnp.zeros((16,), jnp.float32)

        def load_w(w, db, va, vb, sd, sv):
            pltpu.async_copy(dst_hbm.at[w0 + w], db, sd)
            ecol = (w0 + w) * EPW
            pltpu.async_copy(eaT_hbm.at[fa, pl.ds(ecol, EPW)], va, sv)
            pltpu.async_copy(eaT_hbm.at[fa + 8, pl.ds(ecol, EPW)], vb, sv)

        def wait_w(db, va, vb, sd, sv):
            pltpu.make_async_copy(dst_hbm.at[0], db, sd).wait()
            pltpu.make_async_copy(eaT_hbm.at[0, pl.ds(0, EPW)], va, sv).wait()
            pltpu.make_async_copy(eaT_hbm.at[0, pl.ds(0, EPW)], vb, sv).wait()

        def accumulate(db, va, vb):
            @pl.loop(0, NCH, step=5)
            def _(r0):
                for dr in range(5):
                    for c5 in range(CH // 16):
                        idx = db[r0 + dr, pl.ds(c5 * 16, 16)]
                        off = (r0 + dr) * CH + c5 * 16
                        plsc.addupdate_scatter(acca, [idx],
                                               va[pl.ds(off, 16)])
                        plsc.addupdate_scatter(accb, [idx],
                                               vb[pl.ds(off, 16)])

        load_w(0, d0, va0, vb0, sd0, sv0)

        @pl.loop(0, NWH // 2)
        def _(t):
            w = 2 * t
            wait_w(d0, va0, vb0, sd0, sv0)
            load_w(w + 1, d1, va1, vb1, sd1, sv1)
            accumulate(d0, va0, vb0)
            wait_w(d1, va1, vb1, sd1, sv1)

            @pl.when(w + 2 < NWH)
            def _():
                load_w(w + 2, d0, va0, vb0, sd0, sv0)

            accumulate(d1, va1, vb1)

        pltpu.sync_copy(acca, out_hbm.at[q, fa])
        pltpu.sync_copy(accb, out_hbm.at[q, fa + 8])

    return k(eaT, dst3)


def _tc_mlp_pool(x, aggx, agge, batch3, W_edge, W1, b1r, W2, b2r):
    def mmd(a, b):
        return lax.dot_general(a, b, (((1,), (0,)), ((), ())),
                               preferred_element_type=jnp.float32)

    def split(a):
        h = a.astype(jnp.bfloat16)
        l = (a - h.astype(jnp.float32)).astype(jnp.bfloat16)
        return h, l

    def mm(a, b):
        ah, al = split(a)
        bh, bl = split(b)
        return (mmd(ah, bl) + mmd(al, bh)) + mmd(ah, bh)

    def body(x_r, a0_r, a1_r, e0_r, e1_r, e2_r, e3_r, we_r, w1_r, b1_r,
             w2_r, b2_r, bat_r, nr_r, gr_r, sums, counts):
        i = pl.program_id(0)

        @pl.when(i == 0)
        def _():
            sums[...] = jnp.zeros_like(sums)
            counts[...] = jnp.zeros_like(counts)

        a16 = (e0_r[0] + e1_r[0]) + (e2_r[0] + e3_r[0])
        h0 = x_r[...] + a0_r[0] + a1_r[0] + mm(a16, we_r[...])
        h = jnp.maximum(mm(h0, w1_r[...]) + b1_r[...], 0.0)
        nr = mm(h, w2_r[...]) + b2_r[...]
        nr_r[...] = nr

        seg = bat_r[0]
        P = (lax.broadcasted_iota(jnp.int32, (G, BN), 0) == seg)
        Pb = P.astype(jnp.bfloat16)
        nh, nl = split(nr)
        sums[...] += mmd(Pb, nh) + mmd(Pb, nl)
        counts[...] += jnp.sum(P.astype(jnp.float32), axis=1, keepdims=True)

        @pl.when(i == NB - 1)
        def _():
            gr_r[...] = sums[...] / jnp.maximum(counts[...], 1.0)

    nr, gr = pl.pallas_call(
        body,
        grid=(NB,),
        in_specs=[
            pl.BlockSpec((BN, D), lambda i: (i, 0)),
            pl.BlockSpec((1, BN, D), lambda i: (0, i, 0)),
            pl.BlockSpec((1, BN, D), lambda i: (1, i, 0)),
            pl.BlockSpec((1, BN, DE), lambda i: (0, i, 0)),
            pl.BlockSpec((1, BN, DE), lambda i: (1, i, 0)),
            pl.BlockSpec((1, BN, DE), lambda i: (2, i, 0)),
            pl.BlockSpec((1, BN, DE), lambda i: (3, i, 0)),
            pl.BlockSpec((DE, D), lambda i: (0, 0)),
            pl.BlockSpec((D, 2 * D), lambda i: (0, 0)),
            pl.BlockSpec((1, 2 * D), lambda i: (0, 0)),
            pl.BlockSpec((2 * D, D), lambda i: (0, 0)),
            pl.BlockSpec((1, D), lambda i: (0, 0)),
            pl.BlockSpec((1, 1, BN), lambda i: (i, 0, 0)),
        ],
        out_specs=[
            pl.BlockSpec((BN, D), lambda i: (i, 0)),
            pl.BlockSpec((G, D), lambda i: (0, 0)),
        ],
        out_shape=[
            jax.ShapeDtypeStruct((N_NODES, D), jnp.float32),
            jax.ShapeDtypeStruct((G, D), jnp.float32),
        ],
        scratch_shapes=[
            pltpu.VMEM((G, D), jnp.float32),
            pltpu.VMEM((G, 1), jnp.float32),
        ],
        compiler_params=pltpu.CompilerParams(
            dimension_semantics=("arbitrary",)),
    )(x, aggx, aggx, agge, agge, agge, agge, W_edge, W1, b1r, W2, b2r, batch3)
    return gr, nr


def kernel(x, edge_index, edge_attr, batch, W_edge, W1, b1, W2, b2):
    src3 = edge_index[0].reshape(NW, NCH, CH)
    dst3 = edge_index[1].reshape(NW, NCH, CH)
    aggx = _sc_segment_sum_x(x, src3, dst3)
    agge = _sc_segment_sum_ea(edge_attr.T, dst3).transpose(0, 2, 1)
    batch3 = batch.reshape(NB, 1, BN)
    gr, nr = _tc_mlp_pool(x, aggx, agge, batch3, W_edge, W1,
                          b1.reshape(1, 2 * D), W2, b2.reshape(1, D))
    return (gr, nr)
